# baseline (speedup 1.0000x reference)
import functools

import jax
import jax.numpy as jnp
from jax import lax
from jax.experimental import pallas as pl
from jax.experimental.pallas import tpu as pltpu
from jax.experimental.pallas import tpu_sc as plsc

K = 5
NUM_REF = 100
D = 128
B = 16384
NPAD = 128
BLOCK_B = 1024

_INFO = plsc.get_sparse_core_info()
NW = _INFO.num_cores * _INFO.num_subcores
LANES = _INFO.num_lanes
ROWS_W = B // NW
GROUPS = ROWS_W // LANES


def _scores_body(x_ref, r_ref, rn_ref, out_ref):
    x = x_ref[...]
    r = r_ref[...]
    d = lax.dot_general(r, x, (((1,), (1,)), ((), ())),
                        preferred_element_type=jnp.float32,
                        precision=lax.Precision.HIGHEST)
    d = jnp.pad(d, ((0, NPAD - NUM_REF), (0, 0)))
    iota = lax.broadcasted_iota(jnp.int32, (NPAD, BLOCK_B), 0)
    out_ref[...] = jnp.where(iota < NUM_REF, rn_ref[...] - 2.0 * d,
                             jnp.float32(3e38))


def _topk_body(s_hbm, out_hbm, sv, out_v):
    wid = lax.axis_index("s") * _INFO.num_cores + lax.axis_index("c")
    base = wid * ROWS_W
    pltpu.sync_copy(s_hbm.at[:, pl.ds(base, ROWS_W)], sv)

    def swap(va, ia, vb, ib):
        cond = vb < va
        return (jnp.where(cond, vb, va), jnp.where(cond, ib, ia),
                jnp.where(cond, va, vb), jnp.where(cond, ia, ib))

    def insert(st, c, ji):
        v0, v1, v2, v3, v4, i0, i1, i2, i3, i4 = st
        cond = c < v4
        v4 = jnp.where(cond, c, v4)
        i4 = jnp.where(cond, ji, i4)
        v3, i3, v4, i4 = swap(v3, i3, v4, i4)
        v2, i2, v3, i3 = swap(v2, i2, v3, i3)
        v1, i1, v2, i2 = swap(v1, i1, v2, i2)
        v0, i0, v1, i1 = swap(v0, i0, v1, i1)
        return (v0, v1, v2, v3, v4, i0, i1, i2, i3, i4)

    def group_pair(gp, _):
        col_a = gp * (2 * LANES)
        col_b = col_a + LANES

        def body(j, st2):
            sta, stb = st2
            ca = sv[j, pl.ds(col_a, LANES)]
            cb = sv[j, pl.ds(col_b, LANES)]
            ji = jnp.full((LANES,), 0, jnp.int32) + j
            return (insert(sta, ca, ji), insert(stb, cb, ji))

        big = jnp.full((LANES,), 3.5e38, jnp.float32)
        zero = jnp.full((LANES,), 0, jnp.int32)
        init = (big,) * K + (zero,) * K
        sta, stb = lax.fori_loop(0, NPAD, body, (init, init))
        for k in range(K):
            out_v[k, pl.ds(col_a, LANES)] = sta[K + k]
            out_v[k, pl.ds(col_b, LANES)] = stb[K + k]
        return 0

    lax.fori_loop(0, GROUPS // 2, group_pair, 0)

    pltpu.sync_copy(out_v, out_hbm.at[:, pl.ds(base, ROWS_W)])


@functools.partial(
    pl.kernel,
    out_type=jax.ShapeDtypeStruct((K, B), jnp.int32),
    mesh=plsc.VectorSubcoreMesh(core_axis_name="c", subcore_axis_name="s"),
    scratch_types=[
        pltpu.VMEM((NPAD, ROWS_W), jnp.float32),
        pltpu.VMEM((K, ROWS_W), jnp.int32),
    ],
)
def _sc_topk(s_hbm, out_hbm, sv, out_v):
    _topk_body(s_hbm, out_hbm, sv, out_v)


@jax.jit
def kernel(inputs, reference_points):
    rn = jnp.sum(reference_points * reference_points, axis=1)
    rn_col = jnp.pad(rn, (0, NPAD - NUM_REF))[:, None]
    grid = B // BLOCK_B
    scores_t = pl.pallas_call(
        _scores_body,
        grid=(grid,),
        in_specs=[
            pl.BlockSpec((BLOCK_B, D), lambda i: (i, 0)),
            pl.BlockSpec((NUM_REF, D), lambda i: (0, 0)),
            pl.BlockSpec((NPAD, 1), lambda i: (0, 0)),
        ],
        out_specs=pl.BlockSpec((NPAD, BLOCK_B), lambda i: (0, i)),
        out_shape=jax.ShapeDtypeStruct((NPAD, B), jnp.float32),
    )(inputs, reference_points, rn_col)
    return _sc_topk(scores_t).T


# speedup vs baseline: 1.1367x; 1.1367x over previous
import functools

import jax
import jax.numpy as jnp
from jax import lax
from jax.experimental import pallas as pl
from jax.experimental.pallas import tpu as pltpu
from jax.experimental.pallas import tpu_sc as plsc

K = 5
NUM_REF = 100
D = 128
B = 16384
NPAD = 128
BLOCK_B = 2048
B_SC = 4096

_INFO = plsc.get_sparse_core_info()
NW = _INFO.num_cores * _INFO.num_subcores
LANES = _INFO.num_lanes
ROWS_W = B_SC // NW
GROUPS = ROWS_W // LANES


def _scores_body(x_ref, r_ref, rn_ref, out_ref):
    x = x_ref[...]
    r = r_ref[...]
    d = lax.dot_general(r, x, (((1,), (1,)), ((), ())),
                        preferred_element_type=jnp.float32,
                        precision=lax.Precision.HIGHEST)
    d = jnp.pad(d, ((0, NPAD - NUM_REF), (0, 0)))
    iota = lax.broadcasted_iota(jnp.int32, (NPAD, BLOCK_B), 0)
    out_ref[...] = jnp.where(iota < NUM_REF, rn_ref[...] - 2.0 * d,
                             jnp.float32(3e38))


def _topk_body(s_hbm, out_hbm, sv, out_v):
    wid = lax.axis_index("s") * _INFO.num_cores + lax.axis_index("c")
    base = wid * ROWS_W
    pltpu.sync_copy(s_hbm.at[:, pl.ds(base, ROWS_W)], sv)

    def swap(va, ia, vb, ib):
        cond = vb < va
        return (jnp.where(cond, vb, va), jnp.where(cond, ib, ia),
                jnp.where(cond, va, vb), jnp.where(cond, ia, ib))

    def insert(st, c, ji):
        v0, v1, v2, v3, v4, i0, i1, i2, i3, i4 = st
        cond = c < v4
        v4 = jnp.where(cond, c, v4)
        i4 = jnp.where(cond, ji, i4)
        v3, i3, v4, i4 = swap(v3, i3, v4, i4)
        v2, i2, v3, i3 = swap(v2, i2, v3, i3)
        v1, i1, v2, i2 = swap(v1, i1, v2, i2)
        v0, i0, v1, i1 = swap(v0, i0, v1, i1)
        return (v0, v1, v2, v3, v4, i0, i1, i2, i3, i4)

    for g in range(0, GROUPS, 2):
        col_a = g * LANES
        col_b = col_a + LANES

        def body(j, st2):
            sta, stb = st2
            ca = sv[j, pl.ds(col_a, LANES)]
            cb = sv[j, pl.ds(col_b, LANES)]
            ji = jnp.full((LANES,), 0, jnp.int32) + j
            return (insert(sta, ca, ji), insert(stb, cb, ji))

        big = jnp.full((LANES,), 3.5e38, jnp.float32)
        zero = jnp.full((LANES,), 0, jnp.int32)
        init = (big,) * K + (zero,) * K
        sta, stb = lax.fori_loop(0, NPAD, body, (init, init))
        for k in range(K):
            out_v[k, pl.ds(col_a, LANES)] = sta[K + k]
            out_v[k, pl.ds(col_b, LANES)] = stb[K + k]

    pltpu.sync_copy(out_v, out_hbm.at[:, pl.ds(base, ROWS_W)])


@functools.partial(
    pl.kernel,
    out_type=jax.ShapeDtypeStruct((K, B_SC), jnp.int32),
    mesh=plsc.VectorSubcoreMesh(core_axis_name="c", subcore_axis_name="s"),
    scratch_types=[
        pltpu.VMEM((NPAD, ROWS_W), jnp.float32),
        pltpu.VMEM((K, ROWS_W), jnp.int32),
    ],
)
def _sc_topk(s_hbm, out_hbm, sv, out_v):
    _topk_body(s_hbm, out_hbm, sv, out_v)


def _knn_fused_body(x_ref, r_ref, rn_ref, out_ref):
    x = x_ref[...]
    r = r_ref[...]
    d = lax.dot_general(x, r, (((1,), (1,)), ((), ())),
                        preferred_element_type=jnp.float32,
                        precision=lax.Precision.HIGHEST)
    iota = lax.broadcasted_iota(jnp.int32, (BLOCK_B, NPAD), 1).astype(
        jnp.float32)
    pad = jnp.pad(d, ((0, 0), (0, NPAD - NUM_REF)))
    s = jnp.where(iota < float(NUM_REF), rn_ref[...] - 2.0 * pad,
                  jnp.float32(3e38))
    cols = []
    for _ in range(K):
        m = jnp.min(s, axis=1, keepdims=True)
        is_min = s == m
        idx = jnp.min(jnp.where(is_min, iota, jnp.float32(3e38)), axis=1)
        cols.append(idx)
        s = jnp.where(is_min, jnp.float32(jnp.inf), s)
    out_ref[...] = jnp.stack(cols, axis=1).astype(jnp.int32)


@jax.jit
def kernel(inputs, reference_points):
    rn = jnp.sum(reference_points * reference_points, axis=1)
    rn_col = jnp.pad(rn, (0, NPAD - NUM_REF))[:, None]
    rn_row = rn_col.T

    scores_t = pl.pallas_call(
        _scores_body,
        grid=(B_SC // BLOCK_B,),
        in_specs=[
            pl.BlockSpec((BLOCK_B, D), lambda i: (i, 0)),
            pl.BlockSpec((NUM_REF, D), lambda i: (0, 0)),
            pl.BlockSpec((NPAD, 1), lambda i: (0, 0)),
        ],
        out_specs=pl.BlockSpec((NPAD, BLOCK_B), lambda i: (0, i)),
        out_shape=jax.ShapeDtypeStruct((NPAD, B_SC), jnp.float32),
    )(inputs, reference_points, rn_col)
    out_sc = _sc_topk(scores_t)

    n_tc = B - B_SC
    off = B_SC // BLOCK_B
    out_tc = pl.pallas_call(
        _knn_fused_body,
        grid=(n_tc // BLOCK_B,),
        in_specs=[
            pl.BlockSpec((BLOCK_B, D), lambda i: (i + off, 0)),
            pl.BlockSpec((NUM_REF, D), lambda i: (0, 0)),
            pl.BlockSpec((1, NPAD), lambda i: (0, 0)),
        ],
        out_specs=pl.BlockSpec((BLOCK_B, K), lambda i: (i, 0)),
        out_shape=jax.ShapeDtypeStruct((n_tc, K), jnp.int32),
    )(inputs, reference_points, rn_row)

    return jnp.concatenate([out_sc.T, out_tc], axis=0)


# speedup vs baseline: 1.2754x; 1.1220x over previous
import functools

import jax
import jax.numpy as jnp
from jax import lax
from jax.experimental import pallas as pl
from jax.experimental.pallas import tpu as pltpu
from jax.experimental.pallas import tpu_sc as plsc

K = 5
NUM_REF = 100
D = 128
B = 16384
NPAD = 128
BLOCK_B = 2048
B_SC = 8192

_INFO = plsc.get_sparse_core_info()
NW = _INFO.num_cores * _INFO.num_subcores
LANES = _INFO.num_lanes
ROWS_W = B_SC // NW
GROUPS = ROWS_W // LANES


def _scores_body(x_ref, r_ref, rn_ref, out_ref):
    x = x_ref[...]
    r = r_ref[...]
    d = lax.dot_general(r, x, (((1,), (1,)), ((), ())),
                        preferred_element_type=jnp.float32,
                        precision=lax.Precision.HIGHEST)
    d = jnp.pad(d, ((0, NPAD - NUM_REF), (0, 0)))
    iota = lax.broadcasted_iota(jnp.int32, (NPAD, BLOCK_B), 0)
    out_ref[...] = jnp.where(iota < NUM_REF, rn_ref[...] - 2.0 * d,
                             jnp.float32(3e38))


def _topk_body(s_hbm, out_hbm, sv, out_v):
    wid = lax.axis_index("s") * _INFO.num_cores + lax.axis_index("c")
    base = wid * ROWS_W
    pltpu.sync_copy(s_hbm.at[:, pl.ds(base, ROWS_W)], sv)

    def swap(va, ia, vb, ib):
        cond = vb < va
        return (jnp.where(cond, vb, va), jnp.where(cond, ib, ia),
                jnp.where(cond, va, vb), jnp.where(cond, ia, ib))

    def insert(st, c, ji):
        v0, v1, v2, v3, v4, i0, i1, i2, i3, i4 = st
        cond = c < v4
        v4 = jnp.where(cond, c, v4)
        i4 = jnp.where(cond, ji, i4)
        v3, i3, v4, i4 = swap(v3, i3, v4, i4)
        v2, i2, v3, i3 = swap(v2, i2, v3, i3)
        v1, i1, v2, i2 = swap(v1, i1, v2, i2)
        v0, i0, v1, i1 = swap(v0, i0, v1, i1)
        return (v0, v1, v2, v3, v4, i0, i1, i2, i3, i4)

    for g in range(0, GROUPS, 2):
        col_a = g * LANES
        col_b = col_a + LANES

        def body(j, st2):
            sta, stb = st2
            ca = sv[j, pl.ds(col_a, LANES)]
            cb = sv[j, pl.ds(col_b, LANES)]
            ji = jnp.full((LANES,), 0, jnp.int32) + j
            return (insert(sta, ca, ji), insert(stb, cb, ji))

        big = jnp.full((LANES,), 3.5e38, jnp.float32)
        zero = jnp.full((LANES,), 0, jnp.int32)
        init = (big,) * K + (zero,) * K
        sta, stb = lax.fori_loop(0, NPAD, body, (init, init))
        for k in range(K):
            out_v[k, pl.ds(col_a, LANES)] = sta[K + k]
            out_v[k, pl.ds(col_b, LANES)] = stb[K + k]

    pltpu.sync_copy(out_v, out_hbm.at[:, pl.ds(base, ROWS_W)])


@functools.partial(
    pl.kernel,
    out_type=jax.ShapeDtypeStruct((K, B_SC), jnp.int32),
    mesh=plsc.VectorSubcoreMesh(core_axis_name="c", subcore_axis_name="s"),
    scratch_types=[
        pltpu.VMEM((NPAD, ROWS_W), jnp.float32),
        pltpu.VMEM((K, ROWS_W), jnp.int32),
    ],
)
def _sc_topk(s_hbm, out_hbm, sv, out_v):
    _topk_body(s_hbm, out_hbm, sv, out_v)


def _knn_fused_body(x_ref, r_ref, rn_ref, out_ref):
    x = x_ref[...]
    r = r_ref[...]
    d = lax.dot_general(x, r, (((1,), (1,)), ((), ())),
                        preferred_element_type=jnp.float32,
                        precision=lax.Precision.HIGHEST)
    iota = lax.broadcasted_iota(jnp.int32, (BLOCK_B, NPAD), 1).astype(
        jnp.float32)
    pad = jnp.pad(d, ((0, 0), (0, NPAD - NUM_REF)))
    s = jnp.where(iota < float(NUM_REF), rn_ref[...] - 2.0 * pad,
                  jnp.float32(3e38))
    cols = []
    for _ in range(K):
        m = jnp.min(s, axis=1, keepdims=True)
        is_min = s == m
        idx = jnp.min(jnp.where(is_min, iota, jnp.float32(3e38)), axis=1)
        cols.append(idx)
        s = jnp.where(is_min, jnp.float32(jnp.inf), s)
    out_ref[...] = jnp.stack(cols, axis=1).astype(jnp.int32)


@jax.jit
def kernel(inputs, reference_points):
    rn = jnp.sum(reference_points * reference_points, axis=1)
    rn_col = jnp.pad(rn, (0, NPAD - NUM_REF))[:, None]
    rn_row = rn_col.T

    scores_t = pl.pallas_call(
        _scores_body,
        grid=(B_SC // BLOCK_B,),
        in_specs=[
            pl.BlockSpec((BLOCK_B, D), lambda i: (i, 0)),
            pl.BlockSpec((NUM_REF, D), lambda i: (0, 0)),
            pl.BlockSpec((NPAD, 1), lambda i: (0, 0)),
        ],
        out_specs=pl.BlockSpec((NPAD, BLOCK_B), lambda i: (0, i)),
        out_shape=jax.ShapeDtypeStruct((NPAD, B_SC), jnp.float32),
    )(inputs, reference_points, rn_col)
    out_sc = _sc_topk(scores_t)

    n_tc = B - B_SC
    off = B_SC // BLOCK_B
    out_tc = pl.pallas_call(
        _knn_fused_body,
        grid=(n_tc // BLOCK_B,),
        in_specs=[
            pl.BlockSpec((BLOCK_B, D), lambda i: (i + off, 0)),
            pl.BlockSpec((NUM_REF, D), lambda i: (0, 0)),
            pl.BlockSpec((1, NPAD), lambda i: (0, 0)),
        ],
        out_specs=pl.BlockSpec((BLOCK_B, K), lambda i: (i, 0)),
        out_shape=jax.ShapeDtypeStruct((n_tc, K), jnp.int32),
    )(inputs, reference_points, rn_row)

    return jnp.concatenate([out_sc.T, out_tc], axis=0)


# speedup vs baseline: 1.2811x; 1.0045x over previous
import functools

import jax
import jax.numpy as jnp
from jax import lax
from jax.experimental import pallas as pl
from jax.experimental.pallas import tpu as pltpu
from jax.experimental.pallas import tpu_sc as plsc

K = 5
NUM_REF = 100
D = 128
B = 16384
NPAD = 128
BLOCK_B = 2048
B_SC = 8192

_INFO = plsc.get_sparse_core_info()
NW = _INFO.num_cores * _INFO.num_subcores
LANES = _INFO.num_lanes
ROWS_W = B_SC // NW
GROUPS = ROWS_W // LANES


def _scores_body(x_ref, r_ref, rn_ref, out_ref):
    x = x_ref[...]
    r = r_ref[...]
    d = lax.dot_general(r, x, (((1,), (1,)), ((), ())),
                        preferred_element_type=jnp.float32,
                        precision=lax.Precision.HIGHEST)
    out_ref[...] = rn_ref[...] - 2.0 * d


def _topk_body(s_hbm, out_hbm, sv, out_v):
    wid = lax.axis_index("s") * _INFO.num_cores + lax.axis_index("c")
    base = wid * ROWS_W
    pltpu.sync_copy(s_hbm.at[:, pl.ds(base, ROWS_W)], sv)

    def swap(va, ia, vb, ib):
        cond = vb < va
        return (jnp.where(cond, vb, va), jnp.where(cond, ib, ia),
                jnp.where(cond, va, vb), jnp.where(cond, ia, ib))

    def insert(st, c, ji):
        v0, v1, v2, v3, v4, i0, i1, i2, i3, i4 = st
        cond = c < v4
        v4 = jnp.where(cond, c, v4)
        i4 = jnp.where(cond, ji, i4)
        v3, i3, v4, i4 = swap(v3, i3, v4, i4)
        v2, i2, v3, i3 = swap(v2, i2, v3, i3)
        v1, i1, v2, i2 = swap(v1, i1, v2, i2)
        v0, i0, v1, i1 = swap(v0, i0, v1, i1)
        return (v0, v1, v2, v3, v4, i0, i1, i2, i3, i4)

    for g in range(0, GROUPS, 2):
        col_a = g * LANES
        col_b = col_a + LANES

        def body(j, st2):
            sta, stb = st2
            ca = sv[j, pl.ds(col_a, LANES)]
            cb = sv[j, pl.ds(col_b, LANES)]
            ji = jnp.full((LANES,), 0, jnp.int32) + j
            return (insert(sta, ca, ji), insert(stb, cb, ji))

        big = jnp.full((LANES,), 3.5e38, jnp.float32)
        zero = jnp.full((LANES,), 0, jnp.int32)
        init = (big,) * K + (zero,) * K
        sta, stb = lax.fori_loop(0, NUM_REF, body, (init, init))
        for k in range(K):
            out_v[k, pl.ds(col_a, LANES)] = sta[K + k]
            out_v[k, pl.ds(col_b, LANES)] = stb[K + k]

    pltpu.sync_copy(out_v, out_hbm.at[:, pl.ds(base, ROWS_W)])


@functools.partial(
    pl.kernel,
    out_type=jax.ShapeDtypeStruct((K, B_SC), jnp.int32),
    mesh=plsc.VectorSubcoreMesh(core_axis_name="c", subcore_axis_name="s"),
    scratch_types=[
        pltpu.VMEM((NUM_REF, ROWS_W), jnp.float32),
        pltpu.VMEM((K, ROWS_W), jnp.int32),
    ],
)
def _sc_topk(s_hbm, out_hbm, sv, out_v):
    _topk_body(s_hbm, out_hbm, sv, out_v)


def _knn_fused_body(x_ref, r_ref, rn_ref, out_ref):
    x = x_ref[...]
    r = r_ref[...]
    d = lax.dot_general(x, r, (((1,), (1,)), ((), ())),
                        preferred_element_type=jnp.float32,
                        precision=lax.Precision.HIGHEST)
    iota = lax.broadcasted_iota(jnp.int32, (BLOCK_B, NPAD), 1).astype(
        jnp.float32)
    pad = jnp.pad(d, ((0, 0), (0, NPAD - NUM_REF)))
    s = jnp.where(iota < float(NUM_REF), rn_ref[...] - 2.0 * pad,
                  jnp.float32(3e38))
    cols = []
    for _ in range(K):
        m = jnp.min(s, axis=1, keepdims=True)
        is_min = s == m
        idx = jnp.min(jnp.where(is_min, iota, jnp.float32(3e38)), axis=1)
        cols.append(idx)
        s = jnp.where(is_min, jnp.float32(jnp.inf), s)
    out_ref[...] = jnp.stack(cols, axis=1).astype(jnp.int32)


@jax.jit
def kernel(inputs, reference_points):
    rn = jnp.sum(reference_points * reference_points, axis=1)
    rn_col = rn[:, None]
    rn_row = jnp.pad(rn, (0, NPAD - NUM_REF))[None, :]

    scores_t = pl.pallas_call(
        _scores_body,
        grid=(B_SC // BLOCK_B,),
        in_specs=[
            pl.BlockSpec((BLOCK_B, D), lambda i: (i, 0)),
            pl.BlockSpec((NUM_REF, D), lambda i: (0, 0)),
            pl.BlockSpec((NUM_REF, 1), lambda i: (0, 0)),
        ],
        out_specs=pl.BlockSpec((NUM_REF, BLOCK_B), lambda i: (0, i)),
        out_shape=jax.ShapeDtypeStruct((NUM_REF, B_SC), jnp.float32),
    )(inputs, reference_points, rn_col)
    out_sc = _sc_topk(scores_t)

    n_tc = B - B_SC
    off = B_SC // BLOCK_B
    out_tc = pl.pallas_call(
        _knn_fused_body,
        grid=(n_tc // BLOCK_B,),
        in_specs=[
            pl.BlockSpec((BLOCK_B, D), lambda i: (i + off, 0)),
            pl.BlockSpec((NUM_REF, D), lambda i: (0, 0)),
            pl.BlockSpec((1, NPAD), lambda i: (0, 0)),
        ],
        out_specs=pl.BlockSpec((BLOCK_B, K), lambda i: (i, 0)),
        out_shape=jax.ShapeDtypeStruct((n_tc, K), jnp.int32),
    )(inputs, reference_points, rn_row)

    return jnp.concatenate([out_sc.T, out_tc], axis=0)
